# baseline (speedup 1.0000x reference)
import functools

import jax
import jax.numpy as jnp
from jax import lax
from jax.experimental import pallas as pl
from jax.experimental.pallas import tpu as pltpu
from jax.experimental.pallas import tpu_sc as plsc

N = 10000
E = 320000
D = 128
ALPHA = 0.2

NC = 2
NS = 16
NW = NC * NS
CH = 40
G = 25
NG = 10
NP = 10240
NPT = NP // NS
NPA = 10112
NPTA = NPA // NS


def _deg_body(dstv_hbm, out_hbm, dst_v, ones_v, zeros_v, deg_sh, sem):
    c = lax.axis_index("c")
    s = lax.axis_index("s")
    wid = c * NS + s
    for i in range(128 // 16):
        ones_v[pl.ds(i * 16, 16)] = jnp.ones((16,), jnp.float32)
    for i in range(NPT // 16):
        zeros_v[pl.ds(i * 16, 16)] = jnp.zeros((16,), jnp.float32)
    pltpu.sync_copy(zeros_v, deg_sh.at[pl.ds(pl.multiple_of(s * NPT, 8),
                                             NPT)])
    pltpu.async_copy(
        dstv_hbm.at[pl.ds(pl.multiple_of(wid * (E // NW), 8), E // NW)],
        dst_v, sem).wait()
    plsc.subcore_barrier()

    def body(j, carry):
        for k in range(G):
            off = pl.multiple_of((j * G + k) * CH, 8)
            pltpu.sync_copy(ones_v.at[pl.ds(0, CH)],
                            deg_sh.at[dst_v.at[pl.ds(off, CH)]], add=True)
        return carry

    lax.fori_loop(0, NG, body, 0)
    plsc.subcore_barrier()
    pltpu.sync_copy(deg_sh.at[pl.ds(pl.multiple_of(s * NPT, 8), NPT)],
                    out_hbm.at[c, pl.ds(pl.multiple_of(s * NPT, 8), NPT)])


def _agg_body(y_hbm, srcv_hbm, dstv_hbm, out_hbm,
              srcga, srcgb, dstga, dstgb, rows0, rows1, zeros_v, agg_sh,
              gsem0, gsem1, ssem0, ssem1, isema, isemb, zsem):
    c = lax.axis_index("c")
    s = lax.axis_index("s")
    wid = c * NS + s
    ebase = wid * (E // NW)
    for i in range(8):
        for k in range(D // 16):
            zeros_v[i, pl.ds(k * 16, 16)] = jnp.zeros((16,), jnp.float32)
    for j in range(NPTA // 8):
        pltpu.async_copy(
            zeros_v,
            agg_sh.at[pl.ds(pl.multiple_of(s * NPTA + j * 8, 8), 8)], zsem)

    def i_start(grp, srcg, dstg, isem):
        off = pl.multiple_of(ebase + grp * (G * CH), 8)
        pltpu.async_copy(srcv_hbm.at[pl.ds(off, G * CH)], srcg, isem)
        pltpu.async_copy(dstv_hbm.at[pl.ds(off, G * CH)], dstg, isem)

    def i_wait(grp, srcg, dstg, isem):
        off = pl.multiple_of(ebase + grp * (G * CH), 8)
        pltpu.make_async_copy(srcv_hbm.at[pl.ds(off, G * CH)], srcg,
                              isem).wait()
        pltpu.make_async_copy(dstv_hbm.at[pl.ds(off, G * CH)], dstg,
                              isem).wait()

    pltpu.sync_copy(srcv_hbm.at[pl.ds(pl.multiple_of(ebase, 8), G * CH)],
                    srcga)
    pltpu.sync_copy(dstv_hbm.at[pl.ds(pl.multiple_of(ebase, 8), G * CH)],
                    dstga)
    for j in range(NPTA // 8):
        pltpu.make_async_copy(
            zeros_v,
            agg_sh.at[pl.ds(pl.multiple_of(s * NPTA + j * 8, 8), 8)],
            zsem).wait()
    plsc.subcore_barrier()

    def g_start(srcg, k, rows, gsem):
        pltpu.async_copy(y_hbm.at[srcg.at[pl.ds(k * CH, CH)]], rows, gsem)

    def g_wait(srcg, k, rows, gsem):
        pltpu.make_async_copy(y_hbm.at[srcg.at[pl.ds(k * CH, CH)]], rows,
                              gsem).wait()

    def sc_do(dstg, k, rows, ssem):
        idx = dstg.at[pl.ds(k * CH, CH)]
        pltpu.async_copy(rows, agg_sh.at[idx], ssem, add=True)
        pltpu.make_async_copy(rows, agg_sh.at[idx], ssem).wait()

    i_start(1, srcgb, dstgb, isemb)
    g_start(srcga, 0, rows0, gsem0)
    g_start(srcga, 1, rows1, gsem1)

    rows_ = (rows0, rows1)
    gsem_ = (gsem0, gsem1)
    ssem_ = (ssem0, ssem1)

    def body(u, carry):
        more = u < NG // 2 - 1
        for cidx in range(2 * G):
            b = cidx % 2
            srcg, dstg = (srcga, dstga) if cidx < G else (srcgb, dstgb)
            k = cidx % G
            g_wait(srcg, k, rows_[b], gsem_[b])
            sc_do(dstg, k, rows_[b], ssem_[b])
            nxt = cidx + 2
            if nxt == G - 2:
                i_wait(2 * u + 1, srcgb, dstgb, isemb)
            if nxt < 2 * G:
                nsrc = srcga if nxt < G else srcgb
                g_start(nsrc, nxt % G, rows_[b], gsem_[b])
            else:

                @pl.when(more)
                def _():
                    if nxt == 2 * G:
                        i_wait(2 * u + 2, srcga, dstga, isema)
                    g_start(srcga, nxt % G, rows_[b], gsem_[b])

            if cidx == G - 1:

                @pl.when(more)
                def _():
                    i_start(2 * u + 2, srcga, dstga, isema)

            if cidx == 2 * G - 1:

                @pl.when(more)
                def _():
                    i_start(2 * u + 3, srcgb, dstgb, isemb)

        return carry

    lax.fori_loop(0, NG // 2, body, 0)
    plsc.subcore_barrier()

    wchunks = [(k * 128, 128) for k in range(4)] + [(512, NPTA - 512)]
    for off, ln in wchunks:
        row = pl.multiple_of(s * NPTA + off, 8)
        pltpu.async_copy(agg_sh.at[pl.ds(row, ln)],
                         out_hbm.at[c, pl.ds(row, ln)], zsem)
    for off, ln in wchunks:
        row = pl.multiple_of(s * NPTA + off, 8)
        pltpu.make_async_copy(agg_sh.at[pl.ds(row, ln)],
                              out_hbm.at[c, pl.ds(row, ln)], zsem).wait()


@functools.lru_cache(maxsize=None)
def _sc_kernels():
    mesh = plsc.VectorSubcoreMesh(
        core_axis_name="c", subcore_axis_name="s",
        num_cores=NC, num_subcores=NS)
    deg_kernel = pl.kernel(
        _deg_body,
        out_type=jax.ShapeDtypeStruct((NC, NP), jnp.float32),
        mesh=mesh,
        scratch_types=[
            pltpu.VMEM((E // NW,), jnp.int32),
            pltpu.VMEM((128,), jnp.float32),
            pltpu.VMEM((NPT,), jnp.float32),
            pltpu.VMEM_SHARED((NP,), jnp.float32),
            pltpu.SemaphoreType.DMA,
        ],
    )
    agg_kernel = pl.kernel(
        _agg_body,
        out_type=jax.ShapeDtypeStruct((NC, NP, D), jnp.float32),
        mesh=mesh,
        scratch_types=[
            pltpu.VMEM((G * CH,), jnp.int32),
            pltpu.VMEM((G * CH,), jnp.int32),
            pltpu.VMEM((G * CH,), jnp.int32),
            pltpu.VMEM((G * CH,), jnp.int32),
            pltpu.VMEM((CH, D), jnp.float32),
            pltpu.VMEM((CH, D), jnp.float32),
            pltpu.VMEM((8, D), jnp.float32),
            pltpu.VMEM_SHARED((NPA, D), jnp.float32),
            pltpu.SemaphoreType.DMA,
            pltpu.SemaphoreType.DMA,
            pltpu.SemaphoreType.DMA,
            pltpu.SemaphoreType.DMA,
            pltpu.SemaphoreType.DMA,
            pltpu.SemaphoreType.DMA,
            pltpu.SemaphoreType.DMA,
        ],
    )
    return deg_kernel, agg_kernel


def _prep_body(degt_ref, x_ref, dinv_ref, y_ref):
    dinv = lax.rsqrt(degt_ref[:, 0:1] + degt_ref[:, 1:2] + 1.0)
    dinv_ref[...] = dinv
    y_ref[...] = x_ref[...] * dinv


def _prep(degt, x):
    blk = 1000
    grid = N // blk
    return pl.pallas_call(
        _prep_body,
        grid=(grid,),
        in_specs=[
            pl.BlockSpec((blk, NC), lambda i: (i, 0)),
            pl.BlockSpec((blk, D), lambda i: (i, 0)),
        ],
        out_specs=[
            pl.BlockSpec((blk, 1), lambda i: (i, 0)),
            pl.BlockSpec((blk, D), lambda i: (i, 0)),
        ],
        out_shape=[
            jax.ShapeDtypeStruct((N, 1), jnp.float32),
            jax.ShapeDtypeStruct((N, D), jnp.float32),
        ],
    )(degt, x)


def _final_body(p_ref, dinv_ref, x0_ref, x_ref, w_ref, o_ref):
    dv = dinv_ref[...]
    xv = x_ref[...]
    agg = (p_ref[0] + p_ref[1]) * dv + xv * (dv * dv)
    h = (1.0 - ALPHA) * agg + ALPHA * x0_ref[...]
    mm = jnp.dot(h, w_ref[...], preferred_element_type=jnp.float32)
    o_ref[...] = xv + jnp.maximum(mm, 0.0)


def _final(parts, dinv, x0, x, W1):
    blk = 1000
    grid = N // blk
    return pl.pallas_call(
        _final_body,
        grid=(grid,),
        in_specs=[
            pl.BlockSpec((NC, blk, D), lambda i: (0, i, 0)),
            pl.BlockSpec((blk, 1), lambda i: (i, 0)),
            pl.BlockSpec((blk, D), lambda i: (i, 0)),
            pl.BlockSpec((blk, D), lambda i: (i, 0)),
            pl.BlockSpec((D, D), lambda i: (0, 0)),
        ],
        out_specs=pl.BlockSpec((blk, D), lambda i: (i, 0)),
        out_shape=jax.ShapeDtypeStruct((N, D), jnp.float32),
    )(parts, dinv, x0, x, W1)


def kernel(x, x0, edge_index, W1):
    srcv = edge_index[0]
    dstv = edge_index[1]
    deg_kernel, agg_kernel = _sc_kernels()
    degp = deg_kernel(dstv)
    dinv, y = _prep(degp.T, x)
    parts = agg_kernel(y, srcv, dstv)
    return _final(parts, dinv, x0, x, W1)


# speedup vs baseline: 1.4645x; 1.4645x over previous
import functools

import jax
import jax.numpy as jnp
from jax import lax
from jax.experimental import pallas as pl
from jax.experimental.pallas import tpu as pltpu
from jax.experimental.pallas import tpu_sc as plsc

N = 10000
E = 320000
D = 128
ALPHA = 0.2

NC = 2
NS = 16
NW = NC * NS
CH = 125
G = 8
NG = 10
NP = 10240
NPT = NP // NS
NPA = 10112
NPTA = NPA // NS


def _deg_body(ei_hbm, out_hbm, dst_v, ones_v, zeros_v, deg_sh, sem, sem2):
    c = lax.axis_index("c")
    s = lax.axis_index("s")
    wid = c * NS + s
    for i in range(128 // 16):
        ones_v[pl.ds(i * 16, 16)] = jnp.ones((16,), jnp.float32)
    for i in range(NPT // 16):
        zeros_v[pl.ds(i * 16, 16)] = jnp.zeros((16,), jnp.float32)
    pltpu.sync_copy(zeros_v, deg_sh.at[pl.ds(s * NPT, NPT)])
    pltpu.async_copy(ei_hbm.at[1, wid], dst_v, sem).wait()
    plsc.subcore_barrier()

    sems = (sem, sem2)

    def body(j, carry):
        for k in range(G):
            pltpu.async_copy(ones_v.at[pl.ds(0, CH)],
                             deg_sh.at[dst_v.at[j, k]], sems[k % 2],
                             add=True)
            if k >= 2:
                pltpu.make_async_copy(ones_v.at[pl.ds(0, CH)],
                                      deg_sh.at[dst_v.at[j, k - 2]],
                                      sems[k % 2]).wait()
        for k in range(G - 2, G):
            pltpu.make_async_copy(ones_v.at[pl.ds(0, CH)],
                                  deg_sh.at[dst_v.at[j, k]],
                                  sems[k % 2]).wait()
        return carry

    lax.fori_loop(0, NG, body, 0)
    plsc.subcore_barrier()
    pltpu.sync_copy(deg_sh.at[pl.ds(s * NPT, NPT)],
                    out_hbm.at[c, pl.ds(s * NPT, NPT)])


def _agg_body(y_hbm, ei_hbm, out_hbm,
              srcga, srcgb, dstga, dstgb, rows0, rows1, zeros_v, agg_sh,
              gsem0, gsem1, ssem0, ssem1, isema, isemb, zsem):
    c = lax.axis_index("c")
    s = lax.axis_index("s")
    wid = c * NS + s
    for i in range(8):
        for k in range(D // 16):
            zeros_v[i, pl.ds(k * 16, 16)] = jnp.zeros((16,), jnp.float32)
    for j in range(NPTA // 8):
        pltpu.async_copy(zeros_v, agg_sh.at[pl.ds(s * NPTA + j * 8, 8)],
                         zsem)
    pltpu.sync_copy(ei_hbm.at[0, wid, 0], srcga)
    pltpu.sync_copy(ei_hbm.at[1, wid, 0], dstga)
    for j in range(NPTA // 8):
        pltpu.make_async_copy(zeros_v,
                              agg_sh.at[pl.ds(s * NPTA + j * 8, 8)],
                              zsem).wait()
    plsc.subcore_barrier()

    def i_start(grp, srcg, dstg, isem):
        pltpu.async_copy(ei_hbm.at[0, wid, grp], srcg, isem)
        pltpu.async_copy(ei_hbm.at[1, wid, grp], dstg, isem)

    def i_wait(grp, srcg, dstg, isem):
        pltpu.make_async_copy(ei_hbm.at[0, wid, grp], srcg, isem).wait()
        pltpu.make_async_copy(ei_hbm.at[1, wid, grp], dstg, isem).wait()

    def g_start(srcg, k, rows, gsem):
        pltpu.async_copy(y_hbm.at[srcg.at[k]], rows, gsem)

    def g_wait(srcg, k, rows, gsem):
        pltpu.make_async_copy(y_hbm.at[srcg.at[k]], rows, gsem).wait()

    def sc_do(dstg, k, rows, ssem):
        pltpu.async_copy(rows, agg_sh.at[dstg.at[k]], ssem, add=True)
        pltpu.make_async_copy(rows, agg_sh.at[dstg.at[k]], ssem).wait()

    i_start(1, srcgb, dstgb, isemb)
    g_start(srcga, 0, rows0, gsem0)
    g_start(srcga, 1, rows1, gsem1)

    rows_ = (rows0, rows1)
    gsem_ = (gsem0, gsem1)
    ssem_ = (ssem0, ssem1)

    def body(u, carry):
        more = u < NG // 2 - 1
        for cidx in range(2 * G):
            b = cidx % 2
            srcg, dstg = (srcga, dstga) if cidx < G else (srcgb, dstgb)
            k = cidx % G
            g_wait(srcg, k, rows_[b], gsem_[b])
            sc_do(dstg, k, rows_[b], ssem_[b])
            nxt = cidx + 2
            if nxt == G - 2:
                i_wait(2 * u + 1, srcgb, dstgb, isemb)
            if nxt < 2 * G:
                nsrc = srcga if nxt < G else srcgb
                g_start(nsrc, nxt % G, rows_[b], gsem_[b])
            else:

                @pl.when(more)
                def _():
                    if nxt == 2 * G:
                        i_wait(2 * u + 2, srcga, dstga, isema)
                    g_start(srcga, nxt % G, rows_[b], gsem_[b])

            if cidx == G - 1:

                @pl.when(more)
                def _():
                    i_start(2 * u + 2, srcga, dstga, isema)

            if cidx == 2 * G - 1:

                @pl.when(more)
                def _():
                    i_start(2 * u + 3, srcgb, dstgb, isemb)

        return carry

    lax.fori_loop(0, NG // 2, body, 0)
    plsc.subcore_barrier()

    wchunks = [(k * 128, 128) for k in range(4)] + [(512, NPTA - 512)]
    for off, ln in wchunks:
        pltpu.async_copy(agg_sh.at[pl.ds(s * NPTA + off, ln)],
                         out_hbm.at[c, pl.ds(s * NPTA + off, ln)], zsem)
    for off, ln in wchunks:
        pltpu.make_async_copy(agg_sh.at[pl.ds(s * NPTA + off, ln)],
                              out_hbm.at[c, pl.ds(s * NPTA + off, ln)],
                              zsem).wait()


@functools.lru_cache(maxsize=None)
def _sc_kernels():
    mesh = plsc.VectorSubcoreMesh(
        core_axis_name="c", subcore_axis_name="s",
        num_cores=NC, num_subcores=NS)
    deg_kernel = pl.kernel(
        _deg_body,
        out_type=jax.ShapeDtypeStruct((NC, NP), jnp.float32),
        mesh=mesh,
        scratch_types=[
            pltpu.VMEM((NG, G, CH), jnp.int32),
            pltpu.VMEM((128,), jnp.float32),
            pltpu.VMEM((NPT,), jnp.float32),
            pltpu.VMEM_SHARED((NP,), jnp.float32),
            pltpu.SemaphoreType.DMA,
            pltpu.SemaphoreType.DMA,
        ],
    )
    agg_kernel = pl.kernel(
        _agg_body,
        out_type=jax.ShapeDtypeStruct((NC, NP, D), jnp.float32),
        mesh=mesh,
        scratch_types=[
            pltpu.VMEM((G, CH), jnp.int32),
            pltpu.VMEM((G, CH), jnp.int32),
            pltpu.VMEM((G, CH), jnp.int32),
            pltpu.VMEM((G, CH), jnp.int32),
            pltpu.VMEM((CH, D), jnp.float32),
            pltpu.VMEM((CH, D), jnp.float32),
            pltpu.VMEM((8, D), jnp.float32),
            pltpu.VMEM_SHARED((NPA, D), jnp.float32),
            pltpu.SemaphoreType.DMA,
            pltpu.SemaphoreType.DMA,
            pltpu.SemaphoreType.DMA,
            pltpu.SemaphoreType.DMA,
            pltpu.SemaphoreType.DMA,
            pltpu.SemaphoreType.DMA,
            pltpu.SemaphoreType.DMA,
        ],
    )
    return deg_kernel, agg_kernel


def _prep_body(degt_ref, x_ref, dinv_ref, y_ref):
    dinv = lax.rsqrt(degt_ref[:, 0:1] + degt_ref[:, 1:2] + 1.0)
    dinv_ref[...] = dinv
    y_ref[...] = x_ref[...] * dinv


def _prep(degt, x):
    blk = 2000
    grid = N // blk
    return pl.pallas_call(
        _prep_body,
        grid=(grid,),
        in_specs=[
            pl.BlockSpec((blk, NC), lambda i: (i, 0)),
            pl.BlockSpec((blk, D), lambda i: (i, 0)),
        ],
        out_specs=[
            pl.BlockSpec((blk, 1), lambda i: (i, 0)),
            pl.BlockSpec((blk, D), lambda i: (i, 0)),
        ],
        out_shape=[
            jax.ShapeDtypeStruct((N, 1), jnp.float32),
            jax.ShapeDtypeStruct((N, D), jnp.float32),
        ],
    )(degt, x)


def _final_body(p_ref, dinv_ref, x0_ref, x_ref, w_ref, o_ref):
    dv = dinv_ref[...]
    xv = x_ref[...]
    agg = (p_ref[0] + p_ref[1]) * dv + xv * (dv * dv)
    h = (1.0 - ALPHA) * agg + ALPHA * x0_ref[...]
    mm = jnp.dot(h, w_ref[...], preferred_element_type=jnp.float32)
    o_ref[...] = xv + jnp.maximum(mm, 0.0)


def _final(parts, dinv, x0, x, W1):
    blk = 2000
    grid = N // blk
    return pl.pallas_call(
        _final_body,
        grid=(grid,),
        in_specs=[
            pl.BlockSpec((NC, blk, D), lambda i: (0, i, 0)),
            pl.BlockSpec((blk, 1), lambda i: (i, 0)),
            pl.BlockSpec((blk, D), lambda i: (i, 0)),
            pl.BlockSpec((blk, D), lambda i: (i, 0)),
            pl.BlockSpec((D, D), lambda i: (0, 0)),
        ],
        out_specs=pl.BlockSpec((blk, D), lambda i: (i, 0)),
        out_shape=jax.ShapeDtypeStruct((N, D), jnp.float32),
    )(parts, dinv, x0, x, W1)


def kernel(x, x0, edge_index, W1):
    ei = edge_index.reshape(2, NW, NG, G, CH)
    deg_kernel, agg_kernel = _sc_kernels()
    degp = deg_kernel(ei)
    dinv, y = _prep(degp.T, x)
    parts = agg_kernel(y, ei)
    return _final(parts, dinv, x0, x, W1)
